# baseline (speedup 1.0000x reference)
import functools

import jax
import jax.numpy as jnp
from jax import lax
from jax.experimental import pallas as pl
from jax.experimental.pallas import tpu as pltpu
from jax.experimental.pallas import tpu_sc as plsc

N = 10000
E = 320000
D = 128
DH = D // 2
NPAD = 10240
NC, NS, L = 2, 16, 16
C = 128
K2 = -(-E // (NS * C))
K2 = K2 + (-K2) % 4
KH = K2 // 2
EPAD = NS * K2 * C
RPT = NPAD // NS

_mesh = plsc.VectorSubcoreMesh(core_axis_name="c", subcore_axis_name="s")
_sc_params = pltpu.CompilerParams(use_tc_tiling_on_sc=False)


def _zero_vmem(buf, rows, cols):
    @pl.loop(0, rows)
    def _(i):
        @pl.loop(0, cols, step=L)
        def _(kk):
            buf[i, pl.ds(kk, L)] = jnp.zeros((L,), jnp.float32)


@functools.partial(
    pl.kernel,
    out_type=jax.ShapeDtypeStruct((NC, NPAD, L), jnp.float32),
    mesh=_mesh,
    scratch_types=[
        pltpu.VMEM((KH, C), jnp.int32),
        pltpu.VMEM((C, L), jnp.float32),
        pltpu.VMEM((C, L), jnp.float32),
        pltpu.VMEM_SHARED((NPAD, L), jnp.float32),
    ],
    compiler_params=_sc_params,
)
def _sc_count(dst_hbm, out_hbm, dst_v, ones_v, zeros_v, acc):
    c = lax.axis_index("c")
    s = lax.axis_index("s")

    @pl.loop(0, C)
    def _(i):
        ones_v[i, pl.ds(0, L)] = jnp.ones((L,), jnp.float32)
    _zero_vmem(zeros_v, C, L)

    @pl.loop(0, RPT, step=C)
    def _(r):
        pltpu.sync_copy(zeros_v, acc.at[pl.ds(s * RPT + r, C)])
    plsc.subcore_barrier()

    pltpu.sync_copy(dst_hbm.at[s].at[pl.ds(c * KH, KH)], dst_v)

    @pl.loop(0, KH)
    def _(j):
        pltpu.sync_copy(ones_v, acc.at[dst_v.at[j]], add=True)
    plsc.subcore_barrier()

    pltpu.sync_copy(acc.at[pl.ds(s * RPT, RPT)],
                    out_hbm.at[c, pl.ds(s * RPT, RPT)])


KB = 40
NB = K2 // KB


@functools.partial(
    pl.kernel,
    out_type=jax.ShapeDtypeStruct((NPAD, D), jnp.float32),
    mesh=_mesh,
    scratch_types=[
        pltpu.VMEM((KB, C), jnp.int32),
        pltpu.VMEM((KB, C), jnp.int32),
        pltpu.VMEM((C, DH), jnp.float32),
        pltpu.VMEM((C, DH), jnp.float32),
        pltpu.VMEM_SHARED((NPAD, DH), jnp.float32),
        pltpu.VMEM_SHARED((NPAD, DH), jnp.float32),
        pltpu.SemaphoreType.DMA,
        pltpu.SemaphoreType.DMA,
    ],
    compiler_params=_sc_params,
)
def _sc_aggregate(g_hbm, src_hbm, dst_hbm, out_hbm,
                  src_v, dst_v, r0, r1, g_s, acc, gs0, gs1):
    c = lax.axis_index("c")
    s = lax.axis_index("s")

    pltpu.sync_copy(g_hbm.at[pl.ds(s * RPT, RPT), pl.ds(c * DH, DH)],
                    g_s.at[pl.ds(s * RPT, RPT)])
    _zero_vmem(r0, C, DH)

    @pl.loop(0, RPT, step=C)
    def _(r):
        pltpu.sync_copy(r0, acc.at[pl.ds(s * RPT + r, C)])
    plsc.subcore_barrier()

    @pl.loop(0, NB)
    def _(blk):
        pltpu.sync_copy(src_hbm.at[s].at[pl.ds(blk * KB, KB)], src_v)
        pltpu.sync_copy(dst_hbm.at[s].at[pl.ds(blk * KB, KB)], dst_v)
        pltpu.async_copy(g_s.at[src_v.at[0]], r0, gs0)

        @pl.loop(0, KB, step=2)
        def _(j):
            pltpu.make_async_copy(g_s.at[src_v.at[j]], r0, gs0).wait()
            pltpu.async_copy(g_s.at[src_v.at[j + 1]], r1, gs1)
            pltpu.sync_copy(r0, acc.at[dst_v.at[j]], add=True)
            pltpu.make_async_copy(g_s.at[src_v.at[j + 1]], r1, gs1).wait()

            @pl.when(j + 2 < KB)
            def _():
                pltpu.async_copy(g_s.at[src_v.at[j + 2]], r0, gs0)
            pltpu.sync_copy(r1, acc.at[dst_v.at[j + 1]], add=True)

    plsc.subcore_barrier()
    pltpu.sync_copy(acc.at[pl.ds(s * RPT, RPT)],
                    out_hbm.at[pl.ds(s * RPT, RPT), pl.ds(c * DH, DH)])


_R = 1024


def _dinv_col(c0_ref, c1_ref):
    return lax.rsqrt(c0_ref[...] + c1_ref[...] + 1.0)


def _layer1_body(x_ref, w_ref, c0_ref, c1_ref, g_ref):
    h = jnp.dot(x_ref[...], w_ref[...], preferred_element_type=jnp.float32)
    g_ref[...] = _dinv_col(c0_ref, c1_ref) * h


def _tc_layer1(x, w, c0, c1):
    return pl.pallas_call(
        _layer1_body,
        grid=(NPAD // _R,),
        in_specs=[
            pl.BlockSpec((_R, D), lambda r: (r, 0)),
            pl.BlockSpec((D, D), lambda r: (0, 0)),
            pl.BlockSpec((_R, 1), lambda r: (r, 0)),
            pl.BlockSpec((_R, 1), lambda r: (r, 0)),
        ],
        out_specs=pl.BlockSpec((_R, D), lambda r: (r, 0)),
        out_shape=jax.ShapeDtypeStruct((NPAD, D), jnp.float32),
    )(x, w, c0, c1)


def _mid_body(p_ref, g_ref, c0_ref, c1_ref, b_ref, w_ref, out_ref):
    dinv = _dinv_col(c0_ref, c1_ref)
    x2 = jnp.maximum(dinv * (p_ref[...] + g_ref[...]) + b_ref[...], 0.0)
    h2 = jnp.dot(x2, w_ref[...], preferred_element_type=jnp.float32)
    out_ref[...] = dinv * h2


def _tc_mid(p, g, c0, c1, b, w):
    return pl.pallas_call(
        _mid_body,
        grid=(NPAD // _R,),
        in_specs=[
            pl.BlockSpec((_R, D), lambda r: (r, 0)),
            pl.BlockSpec((_R, D), lambda r: (r, 0)),
            pl.BlockSpec((_R, 1), lambda r: (r, 0)),
            pl.BlockSpec((_R, 1), lambda r: (r, 0)),
            pl.BlockSpec((1, D), lambda r: (0, 0)),
            pl.BlockSpec((D, D), lambda r: (0, 0)),
        ],
        out_specs=pl.BlockSpec((_R, D), lambda r: (r, 0)),
        out_shape=jax.ShapeDtypeStruct((NPAD, D), jnp.float32),
    )(p, g, c0, c1, b, w)


def _out_body(p_ref, g_ref, c0_ref, c1_ref, b_ref, out_ref):
    dinv = _dinv_col(c0_ref, c1_ref)
    out_ref[...] = dinv * (p_ref[...] + g_ref[...]) + b_ref[...]


def _tc_out(p, g, c0, c1, b):
    return pl.pallas_call(
        _out_body,
        grid=(NPAD // _R,),
        in_specs=[
            pl.BlockSpec((_R, D), lambda r: (r, 0)),
            pl.BlockSpec((_R, D), lambda r: (r, 0)),
            pl.BlockSpec((_R, 1), lambda r: (r, 0)),
            pl.BlockSpec((_R, 1), lambda r: (r, 0)),
            pl.BlockSpec((1, D), lambda r: (0, 0)),
        ],
        out_specs=pl.BlockSpec((_R, D), lambda r: (r, 0)),
        out_shape=jax.ShapeDtypeStruct((N, D), jnp.float32),
    )(p, g, c0, c1, b)


def kernel(x, edge_index, W1, b1, W2, b2):
    src = edge_index[0].astype(jnp.int32)
    dst = edge_index[1].astype(jnp.int32)
    pad_e = EPAD - E
    src_p = jnp.concatenate(
        [src, jnp.zeros((pad_e,), jnp.int32)]).reshape(NS, K2, C)
    dst_p = jnp.concatenate(
        [dst, jnp.full((pad_e,), N, jnp.int32)]).reshape(NS, K2, C)

    x_p = jnp.pad(x, ((0, NPAD - N), (0, 0)))
    b1r = b1.reshape(1, D)
    b2r = b2.reshape(1, D)

    cnt = _sc_count(dst_p)
    c0 = cnt[0, :, 0:1]
    c1 = cnt[1, :, 0:1]

    g1 = _tc_layer1(x_p, W1, c0, c1)
    p1 = _sc_aggregate(g1, src_p, dst_p)
    g2 = _tc_mid(p1, g1, c0, c1, b1r, W2)
    p2 = _sc_aggregate(g2, src_p, dst_p)
    return _tc_out(p2, g2, c0, c1, b2r)


# speedup vs baseline: 1.0218x; 1.0218x over previous
import functools

import jax
import jax.numpy as jnp
from jax import lax
from jax.experimental import pallas as pl
from jax.experimental.pallas import tpu as pltpu
from jax.experimental.pallas import tpu_sc as plsc

N = 10000
E = 320000
D = 128
DH = D // 2
NPAD = 10240
NC, NS, L = 2, 16, 16
C = 128
K2 = -(-E // (NS * C))
K2 = K2 + (-K2) % 4
KH = K2 // 2
EPAD = NS * K2 * C
RPT = NPAD // NS

_mesh = plsc.VectorSubcoreMesh(core_axis_name="c", subcore_axis_name="s")
_sc_params = pltpu.CompilerParams(use_tc_tiling_on_sc=False)


def _zero_vmem(buf, rows, cols):
    @pl.loop(0, rows)
    def _(i):
        @pl.loop(0, cols, step=L)
        def _(kk):
            buf[i, pl.ds(kk, L)] = jnp.zeros((L,), jnp.float32)


@functools.partial(
    pl.kernel,
    out_type=jax.ShapeDtypeStruct((NC, NPAD, L), jnp.float32),
    mesh=_mesh,
    scratch_types=[
        pltpu.VMEM((KH, C), jnp.int32),
        pltpu.VMEM((C, L), jnp.float32),
        pltpu.VMEM((C, L), jnp.float32),
        pltpu.VMEM_SHARED((NPAD, L), jnp.float32),
    ],
    compiler_params=_sc_params,
)
def _sc_count(dst_hbm, out_hbm, dst_v, ones_v, zeros_v, acc):
    c = lax.axis_index("c")
    s = lax.axis_index("s")

    @pl.loop(0, C)
    def _(i):
        ones_v[i, pl.ds(0, L)] = jnp.ones((L,), jnp.float32)
    _zero_vmem(zeros_v, C, L)

    @pl.loop(0, RPT, step=C)
    def _(r):
        pltpu.sync_copy(zeros_v, acc.at[pl.ds(s * RPT + r, C)])
    plsc.subcore_barrier()

    pltpu.sync_copy(dst_hbm.at[s].at[pl.ds(c * KH, KH)], dst_v)

    @pl.loop(0, KH)
    def _(j):
        pltpu.sync_copy(ones_v, acc.at[dst_v.at[j]], add=True)
    plsc.subcore_barrier()

    pltpu.sync_copy(acc.at[pl.ds(s * RPT, RPT)],
                    out_hbm.at[c, pl.ds(s * RPT, RPT)])


KB = 40
NB = K2 // KB


@functools.partial(
    pl.kernel,
    out_type=jax.ShapeDtypeStruct((NPAD, D), jnp.float32),
    mesh=_mesh,
    scratch_types=[
        pltpu.VMEM((KB, C), jnp.int32),
        pltpu.VMEM((KB, C), jnp.int32),
        pltpu.VMEM((C, DH), jnp.float32),
        pltpu.VMEM((C, DH), jnp.float32),
        pltpu.VMEM_SHARED((NPAD, DH), jnp.float32),
        pltpu.VMEM_SHARED((NPAD, DH), jnp.float32),
        pltpu.SemaphoreType.DMA,
        pltpu.SemaphoreType.DMA,
    ],
    compiler_params=_sc_params,
)
def _sc_aggregate(g_hbm, src_hbm, dst_hbm, out_hbm,
                  src_v, dst_v, r0, r1, g_s, acc, gs0, gs1):
    c = lax.axis_index("c")
    s = lax.axis_index("s")

    pltpu.sync_copy(g_hbm.at[pl.ds(s * RPT, RPT), pl.ds(c * DH, DH)],
                    g_s.at[pl.ds(s * RPT, RPT)])
    _zero_vmem(r0, C, DH)

    @pl.loop(0, RPT, step=C)
    def _(r):
        pltpu.sync_copy(r0, acc.at[pl.ds(s * RPT + r, C)])
    plsc.subcore_barrier()

    @pl.loop(0, NB)
    def _(blk):
        pltpu.sync_copy(src_hbm.at[s].at[pl.ds(blk * KB, KB)], src_v)
        pltpu.sync_copy(dst_hbm.at[s].at[pl.ds(blk * KB, KB)], dst_v)
        pltpu.async_copy(g_s.at[src_v.at[0]], r0, gs0)

        @pl.loop(0, KB, step=2)
        def _(j):
            pltpu.make_async_copy(g_s.at[src_v.at[j]], r0, gs0).wait()
            pltpu.async_copy(g_s.at[src_v.at[j + 1]], r1, gs1)
            pltpu.sync_copy(r0, acc.at[dst_v.at[j]], add=True)
            pltpu.make_async_copy(g_s.at[src_v.at[j + 1]], r1, gs1).wait()

            @pl.when(j + 2 < KB)
            def _():
                pltpu.async_copy(g_s.at[src_v.at[j + 2]], r0, gs0)
            pltpu.sync_copy(r1, acc.at[dst_v.at[j + 1]], add=True)

    plsc.subcore_barrier()
    pltpu.sync_copy(acc.at[pl.ds(s * RPT, RPT)],
                    out_hbm.at[pl.ds(s * RPT, RPT), pl.ds(c * DH, DH)])


_R = 1024


def _dinv_col(cnt_blk):
    return lax.rsqrt(cnt_blk[0, :, 0:1] + cnt_blk[1, :, 0:1] + 1.0)


def _layer1_body(x_ref, w_ref, cnt_ref, g_ref):
    h = jnp.dot(x_ref[...], w_ref[...], preferred_element_type=jnp.float32)
    g_ref[...] = _dinv_col(cnt_ref[...]) * h


def _tc_layer1(x, w, cnt):
    return pl.pallas_call(
        _layer1_body,
        grid=(NPAD // _R,),
        in_specs=[
            pl.BlockSpec((_R, D), lambda r: (r, 0)),
            pl.BlockSpec((D, D), lambda r: (0, 0)),
            pl.BlockSpec((NC, _R, L), lambda r: (0, r, 0)),
        ],
        out_specs=pl.BlockSpec((_R, D), lambda r: (r, 0)),
        out_shape=jax.ShapeDtypeStruct((NPAD, D), jnp.float32),
    )(x, w, cnt)


def _mid_body(p_ref, g_ref, cnt_ref, b_ref, w_ref, out_ref):
    dinv = _dinv_col(cnt_ref[...])
    x2 = jnp.maximum(dinv * (p_ref[...] + g_ref[...]) + b_ref[...], 0.0)
    h2 = jnp.dot(x2, w_ref[...], preferred_element_type=jnp.float32)
    out_ref[...] = dinv * h2


def _tc_mid(p, g, cnt, b, w):
    return pl.pallas_call(
        _mid_body,
        grid=(NPAD // _R,),
        in_specs=[
            pl.BlockSpec((_R, D), lambda r: (r, 0)),
            pl.BlockSpec((_R, D), lambda r: (r, 0)),
            pl.BlockSpec((NC, _R, L), lambda r: (0, r, 0)),
            pl.BlockSpec((1, D), lambda r: (0, 0)),
            pl.BlockSpec((D, D), lambda r: (0, 0)),
        ],
        out_specs=pl.BlockSpec((_R, D), lambda r: (r, 0)),
        out_shape=jax.ShapeDtypeStruct((NPAD, D), jnp.float32),
    )(p, g, cnt, b, w)


def _out_body(p_ref, g_ref, cnt_ref, b_ref, out_ref):
    dinv = _dinv_col(cnt_ref[...])
    out_ref[...] = dinv * (p_ref[...] + g_ref[...]) + b_ref[...]


def _tc_out(p, g, cnt, b):
    return pl.pallas_call(
        _out_body,
        grid=(NPAD // _R,),
        in_specs=[
            pl.BlockSpec((_R, D), lambda r: (r, 0)),
            pl.BlockSpec((_R, D), lambda r: (r, 0)),
            pl.BlockSpec((NC, _R, L), lambda r: (0, r, 0)),
            pl.BlockSpec((1, D), lambda r: (0, 0)),
        ],
        out_specs=pl.BlockSpec((_R, D), lambda r: (r, 0)),
        out_shape=jax.ShapeDtypeStruct((N, D), jnp.float32),
    )(p, g, cnt, b)


def kernel(x, edge_index, W1, b1, W2, b2):
    src = edge_index[0].astype(jnp.int32)
    dst = edge_index[1].astype(jnp.int32)
    pad_e = EPAD - E
    src_p = jnp.concatenate(
        [src, jnp.zeros((pad_e,), jnp.int32)]).reshape(NS, K2, C)
    dst_p = jnp.concatenate(
        [dst, jnp.full((pad_e,), N, jnp.int32)]).reshape(NS, K2, C)

    x_p = jnp.pad(x, ((0, NPAD - N), (0, 0)))
    b1r = b1.reshape(1, D)
    b2r = b2.reshape(1, D)

    cnt = _sc_count(dst_p)

    g1 = _tc_layer1(x_p, W1, cnt)
    p1 = _sc_aggregate(g1, src_p, dst_p)
    g2 = _tc_mid(p1, g1, cnt, b1r, W2)
    p2 = _sc_aggregate(g2, src_p, dst_p)
    return _tc_out(p2, g2, cnt, b2r)
